# baseline (speedup 1.0000x reference)
import functools

import jax
import jax.numpy as jnp
from jax import lax
from jax.experimental import pallas as pl
from jax.experimental.pallas import tpu as pltpu
from jax.experimental.pallas import tpu_sc as plsc

_B = 4096
_L = 16
_NW = 32
_RPW = 48
_SC_ROWS = _NW * _RPW
_TC_ROWS = _B - _SC_ROWS
_STAGE = 8
_NSTAGE = _RPW // _STAGE
_TCBLK = 256


def _sc_body(x_hbm, out_hbm, xbuf0, xbuf1, pbuf, sem0, sem1):
    wid = lax.axis_index("s") * 2 + lax.axis_index("c")
    row0 = _TC_ROWS + wid * _RPW
    lane = lax.iota(jnp.int32, _L)
    bufs = (xbuf0, xbuf1)
    sems = (sem0, sem1)

    def stage_src(s):
        return x_hbm.at[pl.ds(row0 + s * _STAGE, _STAGE), :]

    pltpu.async_copy(stage_src(0), xbuf0, sem0)
    pltpu.async_copy(stage_src(1), xbuf1, sem1)

    def process_row(buf, s, r):
        def p1(jj, accs):
            base = pl.multiple_of(jj * 256, 256)
            a = list(accs)
            for u in range(16):
                a[u % 8] = a[u % 8] + buf[r, pl.ds(base + u * _L, _L)]
            return tuple(a)

        accs = lax.fori_loop(
            0, _B // 256, p1, tuple(jnp.zeros((_L,), jnp.float32) for _ in range(8))
        )
        acc = ((accs[0] + accs[1]) + (accs[2] + accs[3])) + (
            (accs[4] + accs[5]) + (accs[6] + accs[7])
        )
        vlast = buf[r, pl.ds(_B - _L, _L)]
        acc = acc - jnp.where(lane == _L - 1, vlast, 0.0)
        d = jnp.sum(acc)
        rv = 1.0 / jnp.broadcast_to(d, (_L,))

        def p2(jj, paccs):
            base = pl.multiple_of(jj * 256, 256)
            a = list(paccs)
            for u in range(16):
                a[u % 8] = a[u % 8] * (1.0 - buf[r, pl.ds(base + u * _L, _L)] * rv)
            return tuple(a)

        paccs = lax.fori_loop(
            0, _B // 256, p2, tuple(jnp.ones((_L,), jnp.float32) for _ in range(8))
        )
        pacc = ((paccs[0] * paccs[1]) * (paccs[2] * paccs[3])) * (
            (paccs[4] * paccs[5]) * (paccs[6] * paccs[7])
        )
        v0 = buf[r, pl.ds(0, _L)]
        px0 = v0 * rv
        fix0 = jnp.where(lane == 0, px0 / (1.0 - px0), 1.0)
        pxl = vlast * rv
        fixl = jnp.where(lane == _L - 1, 1.0 / (1.0 - pxl), 1.0)
        poff = pl.multiple_of((s * _STAGE + r) * _L, _L)
        pbuf[pl.ds(poff, _L)] = pacc * fix0 * fixl

    def group(g, _):
        for b in range(2):
            s = 2 * g + b
            pltpu.make_async_copy(stage_src(s), bufs[b], sems[b]).wait()
            for r in range(_STAGE):
                process_row(bufs[b], s, r)

            @pl.when(g < _NSTAGE // 2 - 1)
            def _():
                pltpu.async_copy(stage_src(s + 2), bufs[b], sems[b])
        return 0

    lax.fori_loop(0, _NSTAGE // 2, group, 0)
    pltpu.sync_copy(pbuf, out_hbm.at[pl.ds(wid * _RPW * _L, _RPW * _L)])


_sc_call = pl.kernel(
    _sc_body,
    out_type=jax.ShapeDtypeStruct((_SC_ROWS * _L,), jnp.float32),
    mesh=plsc.VectorSubcoreMesh(core_axis_name="c", subcore_axis_name="s"),
    compiler_params=pltpu.CompilerParams(
        needs_layout_passes=False, skip_device_barrier=True
    ),
    scratch_types=[
        pltpu.VMEM((_STAGE, _B), jnp.float32),
        pltpu.VMEM((_STAGE, _B), jnp.float32),
        pltpu.VMEM((_RPW * _L,), jnp.float32),
        pltpu.SemaphoreType.DMA,
        pltpu.SemaphoreType.DMA,
    ],
)


def _tc_body(x_ref, out_ref):
    i = pl.program_id(0)
    xb = x_ref[...]
    s = jnp.sum(xb, axis=1)
    d = s - xb[:, _B - 1]
    rinv = 1.0 / d
    t = 1.0 - xb * rinv[:, None]
    col = jax.lax.broadcasted_iota(jnp.int32, t.shape, 1)
    t = jnp.where((col == 0) | (col == _B - 1), 1.0, t)
    p = t[:, 0:128]
    for k in range(1, _B // 128):
        p = p * t[:, k * 128:(k + 1) * 128]
    row = jnp.log(xb[:, 0] * rinv) + jnp.sum(jnp.log(p), axis=1)
    tot = jnp.sum(row)

    @pl.when(i == 0)
    def _init():
        out_ref[...] = jnp.zeros((1, 1), jnp.float32)

    out_ref[...] += tot.reshape(1, 1)


def _combine_body(tcp_ref, p_ref, out_ref):
    z = jnp.sum(jnp.log(p_ref[...])) + tcp_ref[0, 0]
    out_ref[...] = (z * (-1.0 / _B)).reshape(1, 1)


def kernel(x, targets):
    del targets
    p = _sc_call(x)
    tc_part = pl.pallas_call(
        _tc_body,
        grid=(_TC_ROWS // _TCBLK,),
        in_specs=[pl.BlockSpec((_TCBLK, _B), lambda i: (i, 0))],
        out_specs=pl.BlockSpec((1, 1), lambda i: (0, 0)),
        out_shape=jax.ShapeDtypeStruct((1, 1), jnp.float32),
    )(x)
    res = pl.pallas_call(
        _combine_body,
        out_shape=jax.ShapeDtypeStruct((1, 1), jnp.float32),
    )(tc_part, p.reshape(_SC_ROWS // 8, _L * 8))
    return res.reshape(1)


# speedup vs baseline: 1.1329x; 1.1329x over previous
import functools

import jax
import jax.numpy as jnp
from jax import lax
from jax.experimental import pallas as pl
from jax.experimental.pallas import tpu as pltpu
from jax.experimental.pallas import tpu_sc as plsc

_B = 4096
_L = 16
_NW = 32
_RPW = 32
_SC_ROWS = _NW * _RPW
_TC_ROWS = _B - _SC_ROWS
_STAGE = 8
_NSTAGE = _RPW // _STAGE
_TCBLK = 256


def _sc_body(x_hbm, out_hbm, xbuf0, xbuf1, pbuf, sem0, sem1):
    wid = lax.axis_index("s") * 2 + lax.axis_index("c")
    row0 = _TC_ROWS + wid * _RPW
    lane = lax.iota(jnp.int32, _L)
    bufs = (xbuf0, xbuf1)
    sems = (sem0, sem1)

    def stage_src(s):
        return x_hbm.at[pl.ds(row0 + s * _STAGE, _STAGE), :]

    pltpu.async_copy(stage_src(0), xbuf0, sem0)
    pltpu.async_copy(stage_src(1), xbuf1, sem1)

    def process_row(buf, s, r):
        def p1(jj, accs):
            base = pl.multiple_of(jj * 256, 256)
            a = list(accs)
            for u in range(16):
                a[u % 8] = a[u % 8] + buf[r, pl.ds(base + u * _L, _L)]
            return tuple(a)

        accs = lax.fori_loop(
            0, _B // 256, p1, tuple(jnp.zeros((_L,), jnp.float32) for _ in range(8))
        )
        acc = ((accs[0] + accs[1]) + (accs[2] + accs[3])) + (
            (accs[4] + accs[5]) + (accs[6] + accs[7])
        )
        vlast = buf[r, pl.ds(_B - _L, _L)]
        acc = acc - jnp.where(lane == _L - 1, vlast, 0.0)
        d = jnp.sum(acc)
        rv = 1.0 / jnp.broadcast_to(d, (_L,))

        def p2(jj, paccs):
            base = pl.multiple_of(jj * 256, 256)
            a = list(paccs)
            for u in range(16):
                a[u % 8] = a[u % 8] * (1.0 - buf[r, pl.ds(base + u * _L, _L)] * rv)
            return tuple(a)

        paccs = lax.fori_loop(
            0, _B // 256, p2, tuple(jnp.ones((_L,), jnp.float32) for _ in range(8))
        )
        pacc = ((paccs[0] * paccs[1]) * (paccs[2] * paccs[3])) * (
            (paccs[4] * paccs[5]) * (paccs[6] * paccs[7])
        )
        v0 = buf[r, pl.ds(0, _L)]
        px0 = v0 * rv
        fix0 = jnp.where(lane == 0, px0 / (1.0 - px0), 1.0)
        pxl = vlast * rv
        fixl = jnp.where(lane == _L - 1, 1.0 / (1.0 - pxl), 1.0)
        poff = pl.multiple_of((s * _STAGE + r) * _L, _L)
        pbuf[pl.ds(poff, _L)] = pacc * fix0 * fixl

    def group(g, _):
        for b in range(2):
            s = 2 * g + b
            pltpu.make_async_copy(stage_src(s), bufs[b], sems[b]).wait()
            for r in range(_STAGE):
                process_row(bufs[b], s, r)

            @pl.when(g < _NSTAGE // 2 - 1)
            def _():
                pltpu.async_copy(stage_src(s + 2), bufs[b], sems[b])
        return 0

    lax.fori_loop(0, _NSTAGE // 2, group, 0)
    pltpu.sync_copy(pbuf, out_hbm.at[pl.ds(wid * _RPW * _L, _RPW * _L)])


_sc_call = pl.kernel(
    _sc_body,
    out_type=jax.ShapeDtypeStruct((_SC_ROWS * _L,), jnp.float32),
    mesh=plsc.VectorSubcoreMesh(core_axis_name="c", subcore_axis_name="s"),
    compiler_params=pltpu.CompilerParams(needs_layout_passes=False),
    scratch_types=[
        pltpu.VMEM((_STAGE, _B), jnp.float32),
        pltpu.VMEM((_STAGE, _B), jnp.float32),
        pltpu.VMEM((_RPW * _L,), jnp.float32),
        pltpu.SemaphoreType.DMA,
        pltpu.SemaphoreType.DMA,
    ],
)


def _tc_body(x_ref, out_ref):
    i = pl.program_id(0)
    xb = x_ref[...]
    s = jnp.sum(xb, axis=1)
    x0 = xb[:, 0]
    xl = xb[:, _B - 1]
    d = s - xl
    rinv = 1.0 / d
    rc = rinv[:, None]
    p = 1.0 - xb[:, 0:128] * rc
    for k in range(1, _B // 128):
        p = p * (1.0 - xb[:, k * 128:(k + 1) * 128] * rc)
    row = (jnp.log(x0 * rinv) - jnp.log(1.0 - x0 * rinv)
           - jnp.log(1.0 - xl * rinv) + jnp.sum(jnp.log(p), axis=1))
    tot = jnp.sum(row)

    @pl.when(i == 0)
    def _init():
        out_ref[...] = jnp.zeros((1, 1), jnp.float32)

    out_ref[...] += tot.reshape(1, 1)


def _combine_body(tcp_ref, p_ref, out_ref):
    z = jnp.sum(jnp.log(p_ref[...])) + tcp_ref[0, 0]
    out_ref[...] = (z * (-1.0 / _B)).reshape(1, 1)


def kernel(x, targets):
    del targets
    p = _sc_call(x)
    tc_part = pl.pallas_call(
        _tc_body,
        grid=(_TC_ROWS // _TCBLK,),
        in_specs=[pl.BlockSpec((_TCBLK, _B), lambda i: (i, 0))],
        out_specs=pl.BlockSpec((1, 1), lambda i: (0, 0)),
        out_shape=jax.ShapeDtypeStruct((1, 1), jnp.float32),
    )(x)
    res = pl.pallas_call(
        _combine_body,
        out_shape=jax.ShapeDtypeStruct((1, 1), jnp.float32),
    )(tc_part, p.reshape(_SC_ROWS // 8, _L * 8))
    return res.reshape(1)
